# Initial kernel scaffold; baseline (speedup 1.0000x reference)
import jax
import jax.numpy as jnp
from jax.experimental import pallas as pl


def kernel(confidence, predicted_locations, labels, gt_locations):
    raise NotImplementedError("write your pallas kernel here")


# speedup vs baseline: 10.8248x; 10.8248x over previous
import jax
import jax.numpy as jnp
from jax.experimental import pallas as pl
from jax.experimental.pallas import tpu as pltpu

NEG_POS_RATIO = 3
_CHUNK = 2048


def _row_kernel(conf_ref, lab_ref, pred_ref, gt_ref, out_ref, nbg_ref):
    b = pl.program_id(0)

    @pl.when(b == 0)
    def _init():
        out_ref[0] = 0.0
        out_ref[1] = 0.0
        out_ref[2] = 0.0

    P = conf_ref.shape[2]

    npos = 0.0
    ce_pos = 0.0
    bg_neg = 0.0
    sl1_row = 0.0

    for c0 in range(0, P, _CHUNK):
        cw = min(_CHUNK, P - c0)
        sl = pl.ds(c0, cw)
        x = conf_ref[0, :, sl]
        lab = lab_ref[0, :, sl]
        pos = lab > 0
        posf = pos.astype(jnp.float32)

        m = jnp.max(x, axis=0, keepdims=True)
        e = jnp.exp(x - m)
        s = jnp.sum(e, axis=0, keepdims=True)
        lse = m + jnp.log(s)

        x0 = x[0:1, :]
        cls_iota = jax.lax.broadcasted_iota(jnp.int32, x.shape, 0)
        xl = jnp.sum(jnp.where(cls_iota == lab, x, 0.0), axis=0, keepdims=True)

        bg = lse - x0
        ce = lse - xl

        npos += jnp.sum(posf)
        ce_pos += jnp.sum(ce * posf)
        bg_neg += jnp.sum(bg * (1.0 - posf))
        nbg_ref[0:1, sl] = jnp.where(pos, -jnp.inf, bg)

        d = pred_ref[0, :, sl] - gt_ref[0, :, sl]
        ad = jnp.abs(d)
        sl1 = jnp.where(ad < 1.0, 0.5 * d * d, ad - 0.5)
        sl1_row += jnp.sum(sl1 * posf)

    nneg = P - npos
    k = NEG_POS_RATIO * npos

    @pl.when(k >= nneg)
    def _fast():
        out_ref[1] += ce_pos + bg_neg

    @pl.when(k < nneg)
    def _slow():
        negbg = nbg_ref[0:1, :]
        finite = jnp.where(negbg == -jnp.inf, jnp.inf, negbg)
        lo0 = jnp.min(finite) - 1.0
        hi0 = jnp.max(negbg)

        def _bisect(_, carry):
            lo, hi = carry
            mid = 0.5 * (lo + hi)
            c = jnp.sum((negbg > mid).astype(jnp.float32))
            return jnp.where(c > k, mid, lo), jnp.where(c > k, hi, mid)

        lo, hi = jax.lax.fori_loop(0, 48, _bisect, (lo0, hi0))
        sel_hi = negbg > hi
        c1 = jnp.sum(sel_hi.astype(jnp.float32))
        s1 = jnp.sum(jnp.where(sel_hi, negbg, 0.0))
        r = k - c1
        band = jnp.logical_and(negbg <= hi, negbg > lo)
        idx = jax.lax.broadcasted_iota(jnp.int32, band.shape, 1)

        def _ibisect(_, carry):
            jlo, jhi = carry
            jm = (jlo + jhi) // 2
            c = jnp.sum(jnp.logical_and(band, idx < jm).astype(jnp.float32))
            return jnp.where(c <= r, jm, jlo), jnp.where(c <= r, jhi, jm)

        jlo, _ = jax.lax.fori_loop(0, 16, _ibisect, (0, P + 1))
        s2 = jnp.sum(jnp.where(jnp.logical_and(band, idx < jlo), negbg, 0.0))
        out_ref[1] += ce_pos + s1 + s2

    out_ref[0] += sl1_row
    out_ref[2] += npos


@jax.jit
def kernel(confidence, predicted_locations, labels, gt_locations):
    B, P, C = confidence.shape
    conf_t = jnp.swapaxes(confidence, 1, 2)
    pred_t = jnp.swapaxes(predicted_locations, 1, 2)
    gt_t = jnp.swapaxes(gt_locations, 1, 2)
    lab3 = labels.reshape(B, 1, P)
    sums = pl.pallas_call(
        _row_kernel,
        grid=(B,),
        in_specs=[
            pl.BlockSpec((1, C, P), lambda b: (b, 0, 0)),
            pl.BlockSpec((1, 1, P), lambda b: (b, 0, 0)),
            pl.BlockSpec((1, 4, P), lambda b: (b, 0, 0)),
            pl.BlockSpec((1, 4, P), lambda b: (b, 0, 0)),
        ],
        out_specs=pl.BlockSpec(memory_space=pltpu.SMEM),
        out_shape=jax.ShapeDtypeStruct((3,), jnp.float32),
        scratch_shapes=[pltpu.VMEM((8, P), jnp.float32)],
    )(conf_t, lab3, pred_t, gt_t)
    num_pos = sums[2]
    return sums[0] / num_pos, sums[1] / num_pos
